# baseline (speedup 1.0000x reference)
import jax
import jax.numpy as jnp
from jax.experimental import pallas as pl
from jax.experimental.pallas import tpu as pltpu


def _l1_kernel(rows_ref, w_ref, b_ref, o_ref):
    rows = rows_ref[0]
    hp, ap, _ = rows.shape
    kh = w_ref.shape[0]
    ho = hp - kh + 1
    rowsf = rows.reshape(hp * ap, rows.shape[-1])
    acc = None
    for i in range(kh):
        a_i = rowsf[i * ap:i * ap + ho * ap]
        p = jnp.dot(a_i, w_ref[i], preferred_element_type=jnp.float32)
        acc = p if acc is None else acc + p
    acc = jnp.maximum(acc + b_ref[...], 0.0)
    m = jnp.maximum(acc[:, :64], acc[:, 64:])
    y = m.reshape(ho // 2, 2, ap, 64)
    y = jnp.maximum(y[:, 0], y[:, 1])
    o_ref[0] = y[:, :49, :].astype(o_ref.dtype)


def _make_conv_kernel(kh, kw, wpad_r, wop, pool):
    def body(x_ref, w_ref, b_ref, o_ref):
        x = x_ref[0]
        h = x.shape[0]
        x = jnp.pad(x, ((1, 1), (1, wpad_r), (0, 0)))
        hp = h + 2
        ho = hp - kh + 1
        rows = jnp.concatenate(
            [x[:, j:j + wop, :] for j in range(kw)], axis=-1)
        rowsf = rows.reshape(hp * wop, rows.shape[-1])
        acc = None
        for i in range(kh):
            a_i = rowsf[i * wop:i * wop + ho * wop]
            p = jnp.dot(a_i, w_ref[i], preferred_element_type=jnp.float32)
            acc = p if acc is None else acc + p
        acc = jnp.maximum(acc + b_ref[...], 0.0)
        cout = o_ref.shape[-1]
        y = acc.reshape(ho, wop, cout)
        if pool:
            wq = o_ref.shape[2]
            y = y[:2 * (ho // 2)].reshape(ho // 2, 2, wop, cout)
            y = jnp.maximum(y[:, 0], y[:, 1])
            y = y[:, :2 * wq, :].reshape(ho // 2, wq, 2, cout)
            y = jnp.maximum(y[:, :, 0], y[:, :, 1])
        else:
            y = y[:, :o_ref.shape[2], :]
        o_ref[0] = y.astype(o_ref.dtype)
    return body


def _call(body, x, wmat, b2, out_shape):
    n = x.shape[0]
    return pl.pallas_call(
        body,
        out_shape=jax.ShapeDtypeStruct(out_shape, jnp.bfloat16),
        grid=(n,),
        in_specs=[
            pl.BlockSpec((1,) + x.shape[1:], lambda i: (i, 0, 0, 0)),
            pl.BlockSpec(wmat.shape, lambda i: (0, 0, 0)),
            pl.BlockSpec(b2.shape, lambda i: (0, 0)),
        ],
        out_specs=pl.BlockSpec((1,) + out_shape[1:], lambda i: (i, 0, 0, 0)),
        compiler_params=pltpu.CompilerParams(
            dimension_semantics=("parallel",),
            vmem_limit_bytes=40 * 1024 * 1024,
        ),
    )(x, wmat, b2)


def _wmat3(w):
    cout, cin, kh, kw = w.shape
    return jnp.transpose(w, (2, 3, 1, 0)).reshape(kh, kw * cin, cout).astype(
        jnp.bfloat16)


def _conv_layer(x, w, b, wpad_r, wop, pool, out_hw):
    cout = w.shape[0]
    body = _make_conv_kernel(w.shape[2], w.shape[3], wpad_r, wop, pool)
    out_shape = (x.shape[0],) + out_hw + (cout,)
    return _call(body, x, _wmat3(w), b.reshape(1, cout).astype(jnp.float32),
                 out_shape)


@jax.jit
def kernel(x_nchw, w1, b1, w2, b2, w3, b3, w4, b4):
    n = x_nchw.shape[0]

    kh1, kw1 = w1.shape[2], w1.shape[3]
    cin1 = w1.shape[1]
    xp = jnp.pad(x_nchw.astype(jnp.bfloat16),
                 ((0, 0), (0, 0), (1, 1), (1, 15)))
    taps = [xp[:, c, :, p + j:p + j + 112:2]
            for p in (0, 1) for j in range(kw1) for c in range(cin1)]
    rows = jnp.stack(taps, axis=-1)

    w1m = _wmat3(w1)
    w1d = jnp.zeros((kh1, 2 * kw1 * 3, 128), jnp.bfloat16)
    w1d = w1d.at[:, :kw1 * 3, :64].set(w1m).at[:, kw1 * 3:, 64:].set(w1m)
    b1d = jnp.concatenate([b1, b1]).reshape(1, 128).astype(jnp.float32)

    x = _call(_l1_kernel, rows, w1d, b1d, (n, 49, 49, 64))
    x = _conv_layer(x, w2, b2, wpad_r=4, wop=48, pool=True, out_hw=(22, 22))
    x = _conv_layer(x, w3, b3, wpad_r=4, wop=24, pool=True, out_hw=(10, 10))
    x = _conv_layer(x, w4, b4, wpad_r=8, wop=16, pool=False, out_hw=(9, 9))

    x = jnp.transpose(x, (0, 3, 1, 2))
    return x.reshape(n, -1).astype(jnp.float32)


# speedup vs baseline: 7.9486x; 7.9486x over previous
import jax
import jax.numpy as jnp
from jax.experimental import pallas as pl
from jax.experimental.pallas import tpu as pltpu


def _l1_kernel(rows_ref, w_ref, b_ref, o_ref):
    rows = rows_ref[0]
    hp, ap, _ = rows.shape
    kh = w_ref.shape[0]
    ho = hp - kh + 1
    rowsf = rows.reshape(hp * ap, rows.shape[-1])
    acc = None
    for i in range(kh):
        a_i = rowsf[i * ap:i * ap + ho * ap]
        p = jnp.dot(a_i, w_ref[i], preferred_element_type=jnp.float32)
        acc = p if acc is None else acc + p
    acc = jnp.maximum(acc + b_ref[...], 0.0)
    m = jnp.maximum(acc[:, :64], acc[:, 64:])
    y = m.reshape(ho // 2, 2, ap, 64)
    y = jnp.maximum(y[:, 0], y[:, 1])
    o_ref[0] = y[:, :49, :].astype(o_ref.dtype)


def _make_conv_kernel(kh, kw, wpad_r, wop, pool):
    def body(x_ref, w_ref, b_ref, o_ref):
        x = x_ref[0]
        h = x.shape[0]
        x = jnp.pad(x, ((1, 1), (1, wpad_r), (0, 0)))
        hp = h + 2
        ho = hp - kh + 1
        rows = jnp.concatenate(
            [x[:, j:j + wop, :] for j in range(kw)], axis=-1)
        rowsf = rows.reshape(hp * wop, rows.shape[-1])
        acc = None
        for i in range(kh):
            a_i = rowsf[i * wop:i * wop + ho * wop]
            p = jnp.dot(a_i, w_ref[i], preferred_element_type=jnp.float32)
            acc = p if acc is None else acc + p
        acc = jnp.maximum(acc + b_ref[...], 0.0)
        cout = o_ref.shape[-1]
        y = acc.reshape(ho, wop, cout)
        if pool:
            wq = o_ref.shape[2]
            y = y[:2 * (ho // 2)].reshape(ho // 2, 2, wop, cout)
            y = jnp.maximum(y[:, 0], y[:, 1])
            y = y[:, :2 * wq, :].reshape(ho // 2, wq, 2, cout)
            y = jnp.maximum(y[:, :, 0], y[:, :, 1])
        else:
            y = y[:, :o_ref.shape[2], :]
        o_ref[0] = y.astype(o_ref.dtype)
    return body


def _call(body, x, wmat, b2, out_shape):
    n = x.shape[0]
    return pl.pallas_call(
        body,
        out_shape=jax.ShapeDtypeStruct(out_shape, jnp.bfloat16),
        grid=(n,),
        in_specs=[
            pl.BlockSpec((1,) + x.shape[1:], lambda i: (i, 0, 0, 0)),
            pl.BlockSpec(wmat.shape, lambda i: (0, 0, 0)),
            pl.BlockSpec(b2.shape, lambda i: (0, 0)),
        ],
        out_specs=pl.BlockSpec((1,) + out_shape[1:], lambda i: (i, 0, 0, 0)),
        compiler_params=pltpu.CompilerParams(
            dimension_semantics=("parallel",),
            vmem_limit_bytes=40 * 1024 * 1024,
        ),
    )(x, wmat, b2)


def _wmat3(w):
    cout, cin, kh, kw = w.shape
    return jnp.transpose(w, (2, 3, 1, 0)).reshape(kh, kw * cin, cout).astype(
        jnp.bfloat16)


def _conv_layer(x, w, b, wpad_r, wop, pool, out_hw):
    cout = w.shape[0]
    body = _make_conv_kernel(w.shape[2], w.shape[3], wpad_r, wop, pool)
    out_shape = (x.shape[0],) + out_hw + (cout,)
    return _call(body, x, _wmat3(w), b.reshape(1, cout).astype(jnp.float32),
                 out_shape)


@jax.jit
def kernel(x_nchw, w1, b1, w2, b2, w3, b3, w4, b4):
    n = x_nchw.shape[0]

    kh1, kw1 = w1.shape[2], w1.shape[3]
    xp = jnp.pad(x_nchw.astype(jnp.bfloat16),
                 ((0, 0), (0, 0), (1, 1), (1, 16)))
    xpv = xp.reshape(n, 3, 107, 61, 2)
    taps = [xpv[:, :, :, (p + j) // 2:(p + j) // 2 + 56, (p + j) % 2]
            for p in (0, 1) for j in range(kw1)]
    rows = jnp.stack(taps, axis=-1)
    rows = jnp.transpose(rows, (0, 2, 3, 1, 4)).reshape(n, 107, 56, 60)

    w1m = jnp.transpose(w1, (2, 3, 1, 0)).astype(jnp.bfloat16)
    wcj = jnp.transpose(w1m, (0, 2, 1, 3))
    w6 = jnp.zeros((kh1, 3, 2, kw1, 128), jnp.bfloat16)
    w6 = w6.at[:, :, 0, :, :64].set(wcj).at[:, :, 1, :, 64:].set(wcj)
    w1d = w6.reshape(kh1, 2 * kw1 * 3, 128)
    b1d = jnp.concatenate([b1, b1]).reshape(1, 128).astype(jnp.float32)

    x = _call(_l1_kernel, rows, w1d, b1d, (n, 49, 49, 64))
    x = _conv_layer(x, w2, b2, wpad_r=4, wop=48, pool=True, out_hw=(22, 22))
    x = _conv_layer(x, w3, b3, wpad_r=4, wop=24, pool=True, out_hw=(10, 10))
    x = _conv_layer(x, w4, b4, wpad_r=8, wop=16, pool=False, out_hw=(9, 9))

    x = jnp.transpose(x, (0, 3, 1, 2))
    return x.reshape(n, -1).astype(jnp.float32)


# speedup vs baseline: 9.9698x; 1.2543x over previous
import jax
import jax.numpy as jnp
from jax.experimental import pallas as pl
from jax.experimental.pallas import tpu as pltpu


def _l1_kernel(rows_ref, w_ref, b_ref, o_ref):
    rows = rows_ref[0]
    hp, ap, _ = rows.shape
    npairs = w_ref.shape[0]
    ho = hp - 2 * npairs + 1
    rowsf = rows.reshape(hp * ap, rows.shape[-1])
    pair = jnp.concatenate(
        [rowsf[:(hp - 1) * ap], rowsf[ap:]], axis=-1)
    acc = None
    for g in range(npairs):
        a_g = pair[2 * g * ap:2 * g * ap + ho * ap]
        p = jnp.dot(a_g, w_ref[g], preferred_element_type=jnp.float32)
        acc = p if acc is None else acc + p
    acc = jnp.maximum(acc + b_ref[...], 0.0)
    m = jnp.maximum(acc[:, :64], acc[:, 64:])
    y = m.reshape(ho // 2, 2, ap, 64)
    y = jnp.maximum(y[:, 0], y[:, 1])
    o_ref[0] = y[:, :49, :].astype(o_ref.dtype)


def _make_conv_kernel(kh, kw, wpad_r, wop, pool):
    def body(x_ref, w_ref, b_ref, o_ref):
        x = x_ref[0]
        h = x.shape[0]
        x = jnp.pad(x, ((1, 1), (1, wpad_r), (0, 0)))
        hp = h + 2
        ho = hp - kh + 1
        rows = jnp.concatenate(
            [x[:, j:j + wop, :] for j in range(kw)], axis=-1)
        rowsf = rows.reshape(hp * wop, rows.shape[-1])
        acc = None
        for i in range(kh):
            a_i = rowsf[i * wop:i * wop + ho * wop]
            p = jnp.dot(a_i, w_ref[i], preferred_element_type=jnp.float32)
            acc = p if acc is None else acc + p
        acc = jnp.maximum(acc + b_ref[...], 0.0)
        cout = o_ref.shape[-1]
        y = acc.reshape(ho, wop, cout)
        if pool:
            wq = o_ref.shape[2]
            y = y[:2 * (ho // 2)].reshape(ho // 2, 2, wop, cout)
            y = jnp.maximum(y[:, 0], y[:, 1])
            y = y[:, :2 * wq, :].reshape(ho // 2, wq, 2, cout)
            y = jnp.maximum(y[:, :, 0], y[:, :, 1])
        else:
            y = y[:, :o_ref.shape[2], :]
        o_ref[0] = y.astype(o_ref.dtype)
    return body


def _call(body, x, wmat, b2, out_shape):
    n = x.shape[0]
    return pl.pallas_call(
        body,
        out_shape=jax.ShapeDtypeStruct(out_shape, jnp.bfloat16),
        grid=(n,),
        in_specs=[
            pl.BlockSpec((1,) + x.shape[1:], lambda i: (i, 0, 0, 0)),
            pl.BlockSpec(wmat.shape, lambda i: (0, 0, 0)),
            pl.BlockSpec(b2.shape, lambda i: (0, 0)),
        ],
        out_specs=pl.BlockSpec((1,) + out_shape[1:], lambda i: (i, 0, 0, 0)),
        compiler_params=pltpu.CompilerParams(
            dimension_semantics=("parallel",),
            vmem_limit_bytes=40 * 1024 * 1024,
        ),
    )(x, wmat, b2)


def _wmat3(w):
    cout, cin, kh, kw = w.shape
    return jnp.transpose(w, (2, 3, 1, 0)).reshape(kh, kw * cin, cout).astype(
        jnp.bfloat16)


def _conv_layer(x, w, b, wpad_r, wop, pool, out_hw):
    cout = w.shape[0]
    body = _make_conv_kernel(w.shape[2], w.shape[3], wpad_r, wop, pool)
    out_shape = (x.shape[0],) + out_hw + (cout,)
    return _call(body, x, _wmat3(w), b.reshape(1, cout).astype(jnp.float32),
                 out_shape)


@jax.jit
def kernel(x_nchw, w1, b1, w2, b2, w3, b3, w4, b4):
    n = x_nchw.shape[0]

    kh1, kw1 = w1.shape[2], w1.shape[3]
    xp = jnp.pad(x_nchw.astype(jnp.bfloat16),
                 ((0, 0), (0, 0), (1, 1), (1, 16)))
    xpv = xp.reshape(n, 3, 107, 61, 2)
    taps = [xpv[:, :, :, (p + j) // 2:(p + j) // 2 + 56, (p + j) % 2]
            for p in (0, 1) for j in range(kw1)]
    rows = jnp.stack(taps, axis=-1)
    rows = jnp.transpose(rows, (0, 2, 3, 1, 4)).reshape(n, 107, 56, 60)

    w1m = jnp.transpose(w1, (2, 3, 1, 0)).astype(jnp.bfloat16)
    wcj = jnp.transpose(w1m, (0, 2, 1, 3))
    w6 = jnp.zeros((kh1, 3, 2, kw1, 128), jnp.bfloat16)
    w6 = w6.at[:, :, 0, :, :64].set(wcj).at[:, :, 1, :, 64:].set(wcj)
    w6 = w6.reshape(kh1 // 2, 2, 2 * kw1 * 3, 128)
    w1d = jnp.concatenate([w6[:, 0], w6[:, 1]], axis=1)
    b1d = jnp.concatenate([b1, b1]).reshape(1, 128).astype(jnp.float32)

    x = _call(_l1_kernel, rows, w1d, b1d, (n, 49, 49, 64))
    x = _conv_layer(x, w2, b2, wpad_r=4, wop=48, pool=True, out_hw=(22, 22))
    x = _conv_layer(x, w3, b3, wpad_r=4, wop=24, pool=True, out_hw=(10, 10))
    x = _conv_layer(x, w4, b4, wpad_r=8, wop=16, pool=False, out_hw=(9, 9))

    x = jnp.transpose(x, (0, 3, 1, 2))
    return x.reshape(n, -1).astype(jnp.float32)


# speedup vs baseline: 10.3379x; 1.0369x over previous
import jax
import jax.numpy as jnp
from jax.experimental import pallas as pl
from jax.experimental.pallas import tpu as pltpu


def _l1_kernel(rows_ref, w_ref, b_ref, o_ref):
    rows = rows_ref[0]
    hp, ap, _ = rows.shape
    npairs = w_ref.shape[0]
    ho = hp - 2 * npairs + 1
    rowsf = rows.reshape(hp * ap, rows.shape[-1])
    pair = jnp.concatenate(
        [rowsf[:(hp - 1) * ap], rowsf[ap:]], axis=-1)
    acc = None
    for g in range(npairs):
        a_g = pair[2 * g * ap:2 * g * ap + ho * ap]
        p = jnp.dot(a_g, w_ref[g], preferred_element_type=jnp.float32)
        acc = p if acc is None else acc + p
    acc = jnp.maximum(acc + b_ref[...], 0.0)
    m = jnp.maximum(acc[:, :64], acc[:, 64:])
    y = m.reshape(ho // 2, 2, ap, 64)
    y = jnp.maximum(y[:, 0], y[:, 1])
    o_ref[0] = y[:, :49, :].astype(o_ref.dtype)


def _make_conv_kernel(kh, kw, wpad_r, wop, pool, nb):
    def body(x_ref, w_ref, b_ref, o_ref):
        for im in range(nb):
            x = x_ref[im]
            h = x.shape[0]
            x = jnp.pad(x, ((1, 1), (1, wpad_r), (0, 0)))
            hp = h + 2
            ho = hp - kh + 1
            rows = jnp.concatenate(
                [x[:, j:j + wop, :] for j in range(kw)], axis=-1)
            rowsf = rows.reshape(hp * wop, rows.shape[-1])
            acc = None
            for i in range(kh):
                a_i = rowsf[i * wop:i * wop + ho * wop]
                p = jnp.dot(a_i, w_ref[i], preferred_element_type=jnp.float32)
                acc = p if acc is None else acc + p
            acc = jnp.maximum(acc + b_ref[...], 0.0)
            cout = o_ref.shape[-1]
            y = acc.reshape(ho, wop, cout)
            if pool:
                wq = o_ref.shape[2]
                y = y[:2 * (ho // 2)].reshape(ho // 2, 2, wop, cout)
                y = jnp.maximum(y[:, 0], y[:, 1])
                y = y[:, :2 * wq, :].reshape(ho // 2, wq, 2, cout)
                y = jnp.maximum(y[:, :, 0], y[:, :, 1])
            else:
                y = y[:, :o_ref.shape[2], :]
            o_ref[im] = y.astype(o_ref.dtype)
    return body


def _call(body, x, wmat, b2, out_shape, nb=1):
    n = x.shape[0]
    return pl.pallas_call(
        body,
        out_shape=jax.ShapeDtypeStruct(out_shape, jnp.bfloat16),
        grid=(n // nb,),
        in_specs=[
            pl.BlockSpec((nb,) + x.shape[1:], lambda i: (i, 0, 0, 0)),
            pl.BlockSpec(wmat.shape, lambda i: (0, 0, 0)),
            pl.BlockSpec(b2.shape, lambda i: (0, 0)),
        ],
        out_specs=pl.BlockSpec((nb,) + out_shape[1:], lambda i: (i, 0, 0, 0)),
        compiler_params=pltpu.CompilerParams(
            dimension_semantics=("parallel",),
            vmem_limit_bytes=40 * 1024 * 1024,
        ),
    )(x, wmat, b2)


def _wmat3(w):
    cout, cin, kh, kw = w.shape
    return jnp.transpose(w, (2, 3, 1, 0)).reshape(kh, kw * cin, cout).astype(
        jnp.bfloat16)


def _conv_layer(x, w, b, wpad_r, wop, pool, out_hw, nb=1):
    cout = w.shape[0]
    body = _make_conv_kernel(w.shape[2], w.shape[3], wpad_r, wop, pool, nb)
    out_shape = (x.shape[0],) + out_hw + (cout,)
    return _call(body, x, _wmat3(w), b.reshape(1, cout).astype(jnp.float32),
                 out_shape, nb=nb)


@jax.jit
def kernel(x_nchw, w1, b1, w2, b2, w3, b3, w4, b4):
    n = x_nchw.shape[0]

    kh1, kw1 = w1.shape[2], w1.shape[3]
    xp = jnp.pad(x_nchw.astype(jnp.bfloat16),
                 ((0, 0), (0, 0), (1, 1), (1, 16)))
    xpv = xp.reshape(n, 3, 107, 61, 2)
    taps = [xpv[:, :, :, (p + j) // 2:(p + j) // 2 + 56, (p + j) % 2]
            for p in (0, 1) for j in range(kw1)]
    rows = jnp.stack(taps, axis=-1)
    rows = jnp.transpose(rows, (0, 2, 3, 1, 4)).reshape(n, 107, 56, 60)

    w1m = jnp.transpose(w1, (2, 3, 1, 0)).astype(jnp.bfloat16)
    wcj = jnp.transpose(w1m, (0, 2, 1, 3))
    w6 = jnp.zeros((kh1, 3, 2, kw1, 128), jnp.bfloat16)
    w6 = w6.at[:, :, 0, :, :64].set(wcj).at[:, :, 1, :, 64:].set(wcj)
    w6 = w6.reshape(kh1 // 2, 2, 2 * kw1 * 3, 128)
    w1d = jnp.concatenate([w6[:, 0], w6[:, 1]], axis=1)
    b1d = jnp.concatenate([b1, b1]).reshape(1, 128).astype(jnp.float32)

    x = _call(_l1_kernel, rows, w1d, b1d, (n, 49, 49, 64))
    x = _conv_layer(x, w2, b2, wpad_r=4, wop=48, pool=True, out_hw=(22, 22),
                    nb=2)
    x = _conv_layer(x, w3, b3, wpad_r=4, wop=24, pool=True, out_hw=(10, 10),
                    nb=4)
    x = _conv_layer(x, w4, b4, wpad_r=8, wop=16, pool=False, out_hw=(9, 9),
                    nb=4)

    x = jnp.transpose(x, (0, 3, 1, 2))
    return x.reshape(n, -1).astype(jnp.float32)
